# baseline (speedup 1.0000x reference)
import functools

import jax
import jax.numpy as jnp
from jax import lax
from jax.experimental import pallas as pl
from jax.experimental.pallas import tpu as pltpu
from jax.experimental.pallas import tpu_sc as plsc

POSE_NUM = 100000
EMBED_DIM = 6
PAD_DIM = 8
BATCH = 4096

_info = plsc.get_sparse_core_info()
_NC = _info.num_cores
_NS = _info.num_subcores
_NW = _NC * _NS
_BPW = BATCH // _NW

_mesh = plsc.VectorSubcoreMesh(core_axis_name="c", subcore_axis_name="s")


@functools.partial(
    pl.kernel,
    mesh=_mesh,
    out_type=jax.ShapeDtypeStruct((BATCH, PAD_DIM), jnp.float32),
    scratch_types=[
        pltpu.VMEM((_BPW,), jnp.int32),
        pltpu.VMEM((_BPW, PAD_DIM), jnp.float32),
        pltpu.SemaphoreType.DMA,
    ],
    compiler_params=pltpu.CompilerParams(use_tc_tiling_on_sc=False),
)
def _sc_gather(idx_hbm, table_hbm, out_hbm, idx_v, rows_v, sem):
    wid = lax.axis_index("s") * _NC + lax.axis_index("c")
    base = wid * _BPW
    pltpu.sync_copy(idx_hbm.at[pl.ds(base, _BPW)], idx_v)
    pltpu.async_copy(table_hbm.at[idx_v], rows_v, sem).wait()
    pltpu.sync_copy(rows_v, out_hbm.at[pl.ds(base, _BPW)])


def kernel(indices, params):
    padded = jnp.pad(params, ((0, 0), (0, PAD_DIM - EMBED_DIM)))
    out = _sc_gather(indices.astype(jnp.int32), padded)
    return out[:, :EMBED_DIM]


# speedup vs baseline: 4.8371x; 4.8371x over previous
import functools

import jax
import jax.numpy as jnp
from jax import lax
from jax.experimental import pallas as pl
from jax.experimental.pallas import tpu as pltpu
from jax.experimental.pallas import tpu_sc as plsc

POSE_NUM = 100000
EMBED_DIM = 6
BATCH = 4096

_info = plsc.get_sparse_core_info()
_NC = _info.num_cores
_NS = _info.num_subcores
_NL = _info.num_lanes
_NW = _NC * _NS
_BPW = BATCH // _NW
_NCHUNK = _BPW // _NL

_mesh = plsc.VectorSubcoreMesh(core_axis_name="c", subcore_axis_name="s")


@functools.partial(
    pl.kernel,
    mesh=_mesh,
    out_type=jax.ShapeDtypeStruct((EMBED_DIM, BATCH), jnp.float32),
    scratch_types=[
        pltpu.VMEM((_BPW,), jnp.int32),
        pltpu.VMEM((EMBED_DIM, _BPW), jnp.int32),
        pltpu.VMEM((EMBED_DIM, _BPW), jnp.float32),
        pltpu.SemaphoreType.DMA,
    ],
)
def _sc_gather(idx_hbm, flat_hbm, out_hbm, idx_v, idx6_v, col6_v, sem):
    wid = lax.axis_index("s") * _NC + lax.axis_index("c")
    base = wid * _BPW
    pltpu.sync_copy(idx_hbm.at[pl.ds(base, _BPW)], idx_v)
    for j in range(EMBED_DIM):
        for c in range(_NCHUNK):
            x = idx_v[pl.ds(c * _NL, _NL)]
            idx6_v[j, pl.ds(c * _NL, _NL)] = x + j * POSE_NUM
    copies = [
        pltpu.async_copy(flat_hbm.at[idx6_v.at[j]], col6_v.at[j], sem)
        for j in range(EMBED_DIM)
    ]
    for cp in copies:
        cp.wait()
    for j in range(EMBED_DIM):
        pltpu.sync_copy(col6_v.at[j], out_hbm.at[j, pl.ds(base, _BPW)])


def kernel(indices, params):
    flat = jnp.reshape(params.T, (POSE_NUM * EMBED_DIM,))
    out_t = _sc_gather(indices.astype(jnp.int32), flat)
    return out_t.T


# speedup vs baseline: 4.8593x; 1.0046x over previous
import functools

import jax
import jax.numpy as jnp
from jax import lax
from jax.experimental import pallas as pl
from jax.experimental.pallas import tpu as pltpu
from jax.experimental.pallas import tpu_sc as plsc

POSE_NUM = 100000
EMBED_DIM = 6
BATCH = 4096

_info = plsc.get_sparse_core_info()
_NC = 1
_NS = _info.num_subcores
_NL = _info.num_lanes
_NW = _NC * _NS
_BPW = BATCH // _NW
_NSEG = 2
_SEG = _BPW // _NSEG
_NCHUNK = _SEG // _NL

_mesh = plsc.VectorSubcoreMesh(
    core_axis_name="c", subcore_axis_name="s", num_cores=_NC
)


@functools.partial(
    pl.kernel,
    mesh=_mesh,
    out_type=jax.ShapeDtypeStruct((EMBED_DIM, BATCH), jnp.float32),
    scratch_types=[
        pltpu.VMEM((_BPW,), jnp.int32),
        pltpu.VMEM((EMBED_DIM * _NSEG, _SEG), jnp.int32),
        pltpu.VMEM((EMBED_DIM * _NSEG, _SEG), jnp.float32),
        pltpu.SemaphoreType.DMA,
    ],
)
def _sc_gather(idx_hbm, flat_hbm, out_hbm, idx_v, idx6_v, col6_v, sem):
    wid = lax.axis_index("s")
    base = wid * _BPW
    pltpu.sync_copy(idx_hbm.at[pl.ds(base, _BPW)], idx_v)
    for j in range(EMBED_DIM):
        for h in range(_NSEG):
            slot = j * _NSEG + h
            for c in range(_NCHUNK):
                x = idx_v[pl.ds(h * _SEG + c * _NL, _NL)]
                idx6_v[slot, pl.ds(c * _NL, _NL)] = x + j * POSE_NUM
    copies = [
        pltpu.async_copy(flat_hbm.at[idx6_v.at[slot]], col6_v.at[slot], sem)
        for slot in range(EMBED_DIM * _NSEG)
    ]
    for cp in copies:
        cp.wait()
    for slot in range(EMBED_DIM * _NSEG):
        j, h = divmod(slot, _NSEG)
        pltpu.sync_copy(
            col6_v.at[slot], out_hbm.at[j, pl.ds(base + h * _SEG, _SEG)]
        )


def kernel(indices, params):
    flat = jnp.reshape(params.T, (POSE_NUM * EMBED_DIM,))
    out_t = _sc_gather(indices.astype(jnp.int32), flat)
    return out_t.T


# speedup vs baseline: 4.9099x; 1.0104x over previous
import functools

import jax
import jax.numpy as jnp
from jax import lax
from jax.experimental import pallas as pl
from jax.experimental.pallas import tpu as pltpu
from jax.experimental.pallas import tpu_sc as plsc

POSE_NUM = 100000
EMBED_DIM = 6
BATCH = 4096

_info = plsc.get_sparse_core_info()
_NC = _info.num_cores
_NS = _info.num_subcores
_NW = _NC * _NS
_BPW = BATCH // _NW

_mesh = plsc.VectorSubcoreMesh(core_axis_name="c", subcore_axis_name="s")


@functools.partial(
    pl.kernel,
    mesh=_mesh,
    out_type=jax.ShapeDtypeStruct((EMBED_DIM, BATCH), jnp.float32),
    scratch_types=[
        pltpu.VMEM((_BPW,), jnp.int32),
        pltpu.VMEM((EMBED_DIM, _BPW), jnp.float32),
        pltpu.SemaphoreType.DMA,
    ],
)
def _sc_gather(idx_hbm, flat_hbm, out_hbm, idx_v, col6_v, sem):
    wid = lax.axis_index("s") * _NC + lax.axis_index("c")
    base = wid * _BPW
    pltpu.sync_copy(idx_hbm.at[pl.ds(base, _BPW)], idx_v)
    copies = [
        pltpu.async_copy(
            flat_hbm.at[pl.ds(j * POSE_NUM, POSE_NUM)].at[idx_v],
            col6_v.at[j],
            sem,
        )
        for j in range(EMBED_DIM)
    ]
    for cp in copies:
        cp.wait()
    pltpu.sync_copy(col6_v, out_hbm.at[:, pl.ds(base, _BPW)])


def kernel(indices, params):
    flat = jnp.reshape(params.T, (POSE_NUM * EMBED_DIM,))
    out_t = _sc_gather(indices.astype(jnp.int32), flat)
    return out_t.T
